# baseline (speedup 1.0000x reference)
import jax
import jax.numpy as jnp
from jax.experimental import pallas as pl
from jax.experimental.pallas import tpu as pltpu

B, L, D = 16, 4096, 1024


def _take_last_body(slen_ref, x_ref, out_ref, sem):
    for b in range(B):
        row = slen_ref[b] - 1
        pltpu.make_async_copy(
            x_ref.at[b, pl.ds(row, 1)], out_ref.at[pl.ds(b, 1)], sem
        ).start()
    pltpu.make_async_copy(x_ref.at[0, pl.ds(0, B)], out_ref, sem).wait()


_take_last = pl.pallas_call(
    _take_last_body,
    out_shape=jax.ShapeDtypeStruct((B, D), jnp.float32),
    in_specs=[
        pl.BlockSpec(memory_space=pltpu.SMEM),
        pl.BlockSpec(memory_space=pl.ANY),
    ],
    out_specs=pl.BlockSpec((B, D), lambda: (0, 0)),
    scratch_shapes=[
        pltpu.SemaphoreType.DMA,
    ],
)


@jax.jit
def kernel(x, seq_len):
    return _take_last(seq_len, x)
